# baseline (speedup 1.0000x reference)
import dataclasses
import functools

import jax
import jax.numpy as jnp
from jax import lax
from jax.experimental import pallas as pl
from jax.experimental.pallas import tpu as pltpu
from jax.experimental.pallas import tpu_sc as plsc

N0, N1, N2 = 50000, 150000, 100000
C = 128
NG = 8
N_LAYERS = 2

NC, NS = 2, 16
CHUNK = 10240
SUB = CHUNK // NS
SCAN = 1024
BATCH = 128
PAD_ROW = 1 << 30

NP0 = 5 * CHUNK
NP1 = 15 * CHUNK
NP2 = 10 * CHUNK

MM_BLK = 1000


def _compiler_params():
    cp = pltpu.CompilerParams()
    if "needs_layout_passes" in pltpu.CompilerParams.__dataclass_fields__:
        cp = dataclasses.replace(cp, needs_layout_passes=False)
    return cp


def _mm_body(m, sig, x_ref, w_ref, *out_refs):
    x = x_ref[...]
    if sig:
        x = 1.0 / (1.0 + jnp.exp(-x))
    for j in range(m):
        out_refs[j][...] = jnp.dot(x, w_ref[j],
                                   preferred_element_type=jnp.float32)


def _matmul_multi(x, ws, n, sig):
    m = ws.shape[0]
    return pl.pallas_call(
        functools.partial(_mm_body, m, sig),
        grid=(n // MM_BLK,),
        in_specs=[
            pl.BlockSpec((MM_BLK, C), lambda i: (i, 0)),
            pl.BlockSpec((m, C, C), lambda i: (0, 0, 0)),
        ],
        out_specs=[pl.BlockSpec((MM_BLK, C), lambda i: (i, 0))] * m,
        out_shape=[jax.ShapeDtypeStruct((n, C), jnp.float32)] * m,
    )(x, ws)


def _pool_body(x_ref, ids_ref, out_ref):
    @pl.when(pl.program_id(0) == 0)
    def _():
        out_ref[...] = jnp.zeros_like(out_ref)

    x = x_ref[...]
    x = 1.0 / (1.0 + jnp.exp(-x))
    ids = ids_ref[...]
    g = lax.broadcasted_iota(jnp.int32, (MM_BLK, NG), 1)
    onehot = (ids == g).astype(jnp.float32)
    out_ref[...] += lax.dot_general(
        onehot, x, (((0,), (0,)), ((), ())),
        preferred_element_type=jnp.float32)


def _pool(y, ids, n):
    return pl.pallas_call(
        _pool_body,
        grid=(n // MM_BLK,),
        in_specs=[
            pl.BlockSpec((MM_BLK, C), lambda i: (i, 0)),
            pl.BlockSpec((MM_BLK, 1), lambda i: (i, 0)),
        ],
        out_specs=pl.BlockSpec((NG, C), lambda i: (0, 0)),
        out_shape=jax.ShapeDtypeStruct((NG, C), jnp.float32),
    )(y, ids)


def _combine_body(p_ref, rw_ref, rb_ref, o_ref):
    acc = jnp.zeros((NG, 3), jnp.float32)
    for r in range(3):
        acc = acc + jnp.dot(p_ref[r], rw_ref[r],
                            preferred_element_type=jnp.float32)
    o_ref[...] = acc + jnp.sum(rb_ref[...], axis=0, keepdims=True)


def _combine(pooled, rw, rb):
    return pl.pallas_call(
        _combine_body,
        out_shape=jax.ShapeDtypeStruct((NG, 3), jnp.float32),
    )(pooled, rw, rb)


def _pad_to(a, n_pad, fill):
    return jnp.concatenate(
        [a, jnp.full((n_pad - a.shape[0],), fill, a.dtype)])


def _scale_rows(rows, wk_val, b):
    @pl.loop(0, BATCH)
    def _(r):
        sp = plsc.load_gather(
            wk_val, [jnp.full((16,), b * BATCH + r, jnp.int32)])
        rr = rows.at[r]
        for cc in range(8):
            s = pl.ds(cc * 16, 16)
            rr[s] = rr[s] * sp


def _spmm_body(n_chunks, nblks, z_refs, pk_refs, zero_hbm, out_ref,
               acc, zbuf, st0, st1, wk_col, wk_loc, wk_val,
               rows0, rows1, ssem0, ssem1, gsem0, gsem1):
    core = lax.axis_index("c")
    sub = lax.axis_index("s")
    zeros16 = jnp.zeros((16,), jnp.float32)
    izeros16 = jnp.zeros((16,), jnp.int32)

    pltpu.sync_copy(zero_hbm, zbuf)

    @pl.loop(0, (SCAN + 160) // 16)
    def _(j):
        wk_val[pl.ds(j * 16, 16)] = zeros16

    @pl.loop(0, SCAN // 16)
    def _(j):
        wk_col[pl.ds(j * 16, 16)] = izeros16

    @pl.loop(0, SCAN // BATCH)
    def _(j):
        for cc in range(8):
            wk_loc[j, pl.ds(cc * 16, 16)] = izeros16

    wb = sub * SUB

    def compact(st, lo):
        def comp(j, off):
            rv = st[pl.ds(j * 16, 16)]
            cv = st[pl.ds(SCAN + j * 16, 16)]
            vv = plsc.bitcast(st[pl.ds(2 * SCAN + j * 16, 16)], jnp.float32)
            m = (rv >= lo) & (rv < lo + CHUNK)
            mi = m.astype(jnp.int32)
            pos = off + plsc.cumsum(mi) - 1
            plsc.store_scatter(wk_col, [pos], cv, mask=m)
            plsc.store_scatter(wk_val, [pos], vv, mask=m)
            plsc.store_scatter(wk_loc, [pos >> 7, pos & 127], rv - lo, mask=m)
            return off + jnp.sum(mi)

        cnt = lax.fori_loop(0, SCAN // 16, comp, jnp.int32(0))
        zeros16 = jnp.zeros((16,), jnp.float32)
        lane = lax.iota(jnp.int32, 16)
        a = (cnt >> 4) << 4
        for g in range(9):
            idxs = a + g * 16 + lane
            plsc.store_scatter(wk_val, [idxs], zeros16, mask=idxs >= cnt)
        return cnt

    def run_batches(z_h, cnt):
        nb = (cnt + BATCH - 1) // BATCH

        def idx(b):
            return wk_col.at[pl.ds(b * BATCH, BATCH)]

        @pl.when(nb > 0)
        def _():
            pltpu.async_copy(z_h.at[idx(0)], rows0, gsem0)

        def pair(pi, carry):
            b0 = 2 * pi
            b1 = b0 + 1
            pltpu.make_async_copy(z_h.at[idx(b0)], rows0, gsem0).wait()

            @pl.when(b1 < nb)
            def _():
                pltpu.async_copy(z_h.at[idx(b1)], rows1, gsem1)

            _scale_rows(rows0, wk_val, b0)
            pltpu.sync_copy(rows0, acc.at[wk_loc.at[b0]], add=True)

            @pl.when(b1 < nb)
            def _():
                pltpu.make_async_copy(z_h.at[idx(b1)], rows1, gsem1).wait()

                @pl.when(b1 + 1 < nb)
                def _():
                    pltpu.async_copy(z_h.at[idx(b1 + 1)], rows0, gsem0)

                _scale_rows(rows1, wk_val, b1)
                pltpu.sync_copy(rows1, acc.at[wk_loc.at[b1]], add=True)

            return carry

        lax.fori_loop(0, (nb + 1) // 2, pair, jnp.int32(0))

    @pl.loop(0, (n_chunks + 1 - core) // NC)
    def _(ci):
        chunk = ci * NC + core
        lo = chunk * CHUNK

        for t in range(SUB // 32):
            pltpu.sync_copy(zbuf, acc.at[pl.ds(wb + t * 32, 32)])
        plsc.subcore_barrier()

        for li in range(len(nblks)):
            z_h = z_refs[li]
            pk = pk_refs[li]
            nblk = nblks[li]
            ubase = sub * nblk

            def stage(b, st, sem):
                return pltpu.make_async_copy(
                    pk.at[pl.ds((ubase + b) * (3 * SCAN), 3 * SCAN)], st, sem)

            stage(0, st0, ssem0).start()

            def block(b, st, sem, stN, semN):
                stage(b, st, sem).wait()

                @pl.when(b + 1 < nblk)
                def _():
                    stage(b + 1, stN, semN).start()

                run_batches(z_h, compact(st, lo))

            @pl.loop(0, (nblk + 1) // 2)
            def _(p):
                b0 = 2 * p
                block(b0, st0, ssem0, st1, ssem1)

                @pl.when(b0 + 1 < nblk)
                def _():
                    block(b0 + 1, st1, ssem1, st0, ssem0)

        plsc.subcore_barrier()
        for t in range(SUB // 128):
            pltpu.sync_copy(acc.at[pl.ds(wb + t * 128, 128)],
                            out_ref.at[pl.ds(lo + wb + t * 128, 128)])
        plsc.subcore_barrier()


def _spmm_sc(n_out_pad, lists, zero_hbm):
    n_chunks = n_out_pad // CHUNK
    nblks = tuple(l[1] for l in lists)
    nl = len(lists)
    mesh = plsc.VectorSubcoreMesh(core_axis_name="c", subcore_axis_name="s",
                                  num_cores=NC, num_subcores=NS)

    def body(*refs):
        z_refs = refs[0:nl]
        pk_refs = refs[nl:2 * nl]
        zero_ref = refs[2 * nl]
        out_ref = refs[2 * nl + 1]
        scratch = refs[2 * nl + 2:]
        _spmm_body(n_chunks, nblks, z_refs, pk_refs, zero_ref, out_ref,
                   *scratch)

    kern = pl.kernel(
        body,
        out_type=jax.ShapeDtypeStruct((n_out_pad, C), jnp.float32),
        mesh=mesh,
        scratch_types=[
            pltpu.VMEM_SHARED((CHUNK, C), jnp.float32),
            pltpu.VMEM((32, C), jnp.float32),
            pltpu.VMEM((3 * SCAN,), jnp.int32),
            pltpu.VMEM((3 * SCAN,), jnp.int32),
            pltpu.VMEM((SCAN,), jnp.int32),
            pltpu.VMEM((SCAN // BATCH, BATCH), jnp.int32),
            pltpu.VMEM((SCAN + 160,), jnp.float32),
            pltpu.VMEM((BATCH, C), jnp.float32),
            pltpu.VMEM((BATCH, C), jnp.float32),
            pltpu.SemaphoreType.DMA,
            pltpu.SemaphoreType.DMA,
            pltpu.SemaphoreType.DMA,
            pltpu.SemaphoreType.DMA,
        ],
        compiler_params=_compiler_params(),
    )
    args = [l[2] for l in lists] + [l[0] for l in lists] + [zero_hbm]
    return kern(*args)


def kernel(x0, x1, x2,
           inc1_row, inc1_col, inc1_val,
           inc2_row, inc2_col, inc2_val,
           h0_row, h0_col, h0_val,
           h1_row, h1_col, h1_val,
           h2_row, h2_col, h2_val,
           xbel0, xbel1, xbel2,
           W_same, W_l2h, W_h2l, RW, Rb):
    f32, i32 = jnp.float32, jnp.int32
    unit = NS * SCAN

    def pad_list(row, col, val):
        n = row.shape[0]
        n_pad = -(-n // unit) * unit
        r = _pad_to(row.astype(i32), n_pad, PAD_ROW).reshape(-1, 1, SCAN)
        c = _pad_to(col.astype(i32), n_pad, 0).reshape(-1, 1, SCAN)
        v = lax.bitcast_convert_type(
            _pad_to(val.astype(f32), n_pad, 0.0), i32).reshape(-1, 1, SCAN)
        packed = jnp.concatenate([r, c, v], axis=1).reshape(-1)
        return (packed, n_pad // (NS * SCAN))

    h0 = pad_list(h0_row, h0_col, h0_val)
    h1 = pad_list(h1_row, h1_col, h1_val)
    h2 = pad_list(h2_row, h2_col, h2_val)
    i1 = pad_list(inc1_row, inc1_col, inc1_val)
    i1t = pad_list(inc1_col, inc1_row, inc1_val)
    i2 = pad_list(inc2_row, inc2_col, inc2_val)
    i2t = pad_list(inc2_col, inc2_row, inc2_val)

    zero_hbm = jnp.zeros((32, C), f32)

    a0, a1, a2 = x0, x1, x2
    for l in range(N_LAYERS):
        sig = l > 0
        z0s, z0l = _matmul_multi(a0, jnp.stack([W_same[l, 0], W_l2h[l, 0]]),
                                 N0, sig)
        z1s, z1h, z1l = _matmul_multi(
            a1, jnp.stack([W_same[l, 1], W_h2l[l, 0], W_l2h[l, 1]]), N1, sig)
        z2s, z2h = _matmul_multi(a2, jnp.stack([W_same[l, 2], W_h2l[l, 1]]),
                                 N2, sig)

        a0 = _spmm_sc(NP0, [h0 + (z0s,), i1 + (z1h,)], zero_hbm)
        a1 = _spmm_sc(NP1, [h1 + (z1s,), i2 + (z2h,), i1t + (z0l,)], zero_hbm)
        a2 = _spmm_sc(NP2, [h2 + (z2s,), i2t + (z1l,)], zero_hbm)

    p0 = _pool(a0, xbel0.astype(i32).reshape(N0, 1), N0)
    p1 = _pool(a1, xbel1.astype(i32).reshape(N1, 1), N1)
    p2 = _pool(a2, xbel2.astype(i32).reshape(N2, 1), N2)

    return _combine(jnp.stack([p0, p1, p2]), RW.astype(f32), Rb.astype(f32))


# speedup vs baseline: 6.1673x; 6.1673x over previous
import dataclasses
import functools

import jax
import jax.numpy as jnp
from jax import lax
from jax.experimental import pallas as pl
from jax.experimental.pallas import tpu as pltpu
from jax.experimental.pallas import tpu_sc as plsc

N0, N1, N2 = 50000, 150000, 100000
C = 128
NG = 8
N_LAYERS = 2

NC, NS = 2, 16
CHUNK = 10240
SUB = CHUNK // NS
SCAN = 1024
BATCH = 128
PAD_ROW = 1 << 30

NP0 = 5 * CHUNK
NP1 = 15 * CHUNK
NP2 = 10 * CHUNK

MM_BLK = 1000


def _compiler_params():
    cp = pltpu.CompilerParams()
    if "needs_layout_passes" in pltpu.CompilerParams.__dataclass_fields__:
        cp = dataclasses.replace(cp, needs_layout_passes=False)
    return cp


def _mm_body(m, sig, x_ref, w_ref, *out_refs):
    x = x_ref[...]
    if sig:
        x = 1.0 / (1.0 + jnp.exp(-x))
    for j in range(m):
        out_refs[j][...] = jnp.dot(x, w_ref[j],
                                   preferred_element_type=jnp.float32)


def _matmul_multi(x, ws, n, sig):
    m = ws.shape[0]
    return pl.pallas_call(
        functools.partial(_mm_body, m, sig),
        grid=(n // MM_BLK,),
        in_specs=[
            pl.BlockSpec((MM_BLK, C), lambda i: (i, 0)),
            pl.BlockSpec((m, C, C), lambda i: (0, 0, 0)),
        ],
        out_specs=[pl.BlockSpec((MM_BLK, C), lambda i: (i, 0))] * m,
        out_shape=[jax.ShapeDtypeStruct((n, C), jnp.float32)] * m,
    )(x, ws)


def _pool_body(x_ref, ids_ref, out_ref):
    @pl.when(pl.program_id(0) == 0)
    def _():
        out_ref[...] = jnp.zeros_like(out_ref)

    x = x_ref[...]
    x = 1.0 / (1.0 + jnp.exp(-x))
    ids = ids_ref[...]
    g = lax.broadcasted_iota(jnp.int32, (MM_BLK, NG), 1)
    onehot = (ids == g).astype(jnp.float32)
    out_ref[...] += lax.dot_general(
        onehot, x, (((0,), (0,)), ((), ())),
        preferred_element_type=jnp.float32)


def _pool(y, ids, n):
    return pl.pallas_call(
        _pool_body,
        grid=(n // MM_BLK,),
        in_specs=[
            pl.BlockSpec((MM_BLK, C), lambda i: (i, 0)),
            pl.BlockSpec((MM_BLK, 1), lambda i: (i, 0)),
        ],
        out_specs=pl.BlockSpec((NG, C), lambda i: (0, 0)),
        out_shape=jax.ShapeDtypeStruct((NG, C), jnp.float32),
    )(y, ids)


def _combine_body(p_ref, rw_ref, rb_ref, o_ref):
    acc = jnp.zeros((NG, 3), jnp.float32)
    for r in range(3):
        acc = acc + jnp.dot(p_ref[r], rw_ref[r],
                            preferred_element_type=jnp.float32)
    o_ref[...] = acc + jnp.sum(rb_ref[...], axis=0, keepdims=True)


def _combine(pooled, rw, rb):
    return pl.pallas_call(
        _combine_body,
        out_shape=jax.ShapeDtypeStruct((NG, 3), jnp.float32),
    )(pooled, rw, rb)


def _pad_to(a, n_pad, fill):
    return jnp.concatenate(
        [a, jnp.full((n_pad - a.shape[0],), fill, a.dtype)])


def _scale_rows(rows, wk_val, b):
    @pl.loop(0, BATCH, step=4)
    def _(r):
        base = b * BATCH + r
        sps = [plsc.load_gather(wk_val,
                                [jnp.full((16,), base + k, jnp.int32)])
               for k in range(4)]
        for k in range(4):
            rr = rows.at[r + k]
            for cc in range(8):
                s = pl.ds(cc * 16, 16)
                rr[s] = rr[s] * sps[k]


WKN = SCAN + BATCH + 32


def _spmm_body(n_chunks, nblks, z_refs, pk_refs, zero_hbm, out_ref,
               acc, zbuf, st0, st1, wk_col, wk_loc, wk_val,
               rows0, rows1, ssem0, ssem1, gsem0, gsem1):
    core = lax.axis_index("c")
    sub = lax.axis_index("s")
    zeros16 = jnp.zeros((16,), jnp.float32)
    izeros16 = jnp.zeros((16,), jnp.int32)
    lane = lax.iota(jnp.int32, 16)

    pltpu.sync_copy(zero_hbm, zbuf)

    @pl.loop(0, WKN // 16)
    def _(j):
        wk_val[pl.ds(j * 16, 16)] = zeros16
        wk_col[pl.ds(j * 16, 16)] = izeros16

    @pl.loop(0, WKN // BATCH + 1)
    def _(j):
        for cc in range(8):
            wk_loc[j, pl.ds(cc * 16, 16)] = izeros16

    wb = sub * SUB

    def compact(st, lo, off0):
        def comp(j, off):
            rv = st[pl.ds(j * 16, 16)]
            cv = st[pl.ds(SCAN + j * 16, 16)]
            vv = plsc.bitcast(st[pl.ds(2 * SCAN + j * 16, 16)], jnp.float32)
            m = (rv >= lo) & (rv < lo + CHUNK)
            mi = m.astype(jnp.int32)
            pos = off + plsc.cumsum(mi) - 1
            plsc.store_scatter(wk_col, [pos], cv, mask=m)
            plsc.store_scatter(wk_val, [pos], vv, mask=m)
            plsc.store_scatter(wk_loc, [pos >> 7, pos & 127], rv - lo, mask=m)
            return off + jnp.sum(mi)

        return lax.fori_loop(0, SCAN // 16, comp, off0)

    def run_batches(z_h, nb):
        def idx(b):
            return wk_col.at[pl.ds(b * BATCH, BATCH)]

        @pl.when(nb > 0)
        def _():
            pltpu.async_copy(z_h.at[idx(0)], rows0, gsem0)

        def pair(pi, carry):
            b0 = 2 * pi
            b1 = b0 + 1
            pltpu.make_async_copy(z_h.at[idx(b0)], rows0, gsem0).wait()

            @pl.when(b1 < nb)
            def _():
                pltpu.async_copy(z_h.at[idx(b1)], rows1, gsem1)

            _scale_rows(rows0, wk_val, b0)
            pltpu.sync_copy(rows0, acc.at[wk_loc.at[b0]], add=True)

            @pl.when(b1 < nb)
            def _():
                pltpu.make_async_copy(z_h.at[idx(b1)], rows1, gsem1).wait()

                @pl.when(b1 + 1 < nb)
                def _():
                    pltpu.async_copy(z_h.at[idx(b1 + 1)], rows0, gsem0)

                _scale_rows(rows1, wk_val, b1)
                pltpu.sync_copy(rows1, acc.at[wk_loc.at[b1]], add=True)

            return carry

        lax.fori_loop(0, (nb + 1) // 2, pair, jnp.int32(0))

    @pl.loop(0, (n_chunks + 1 - core) // NC)
    def _(ci):
        chunk = ci * NC + core
        lo = chunk * CHUNK

        for t in range(SUB // 32):
            pltpu.sync_copy(zbuf, acc.at[pl.ds(wb + t * 32, 32)])
        plsc.subcore_barrier()

        for li in range(len(nblks)):
            z_h = z_refs[li]
            pk = pk_refs[li]
            nblk = nblks[li]
            ubase = sub * nblk

            def stage(b, st, sem):
                return pltpu.make_async_copy(
                    pk.at[pl.ds((ubase + b) * (3 * SCAN), 3 * SCAN)], st, sem)

            def block1(b, st, sem, stN, semN, off):
                stage(b, st, sem).wait()

                @pl.when(b + 1 < nblk)
                def _():
                    stage(b + 1, stN, semN).start()

                off = compact(st, lo, off)
                full = off >> 7
                run_batches(z_h, full)

                @pl.when(full > 0)
                def _():
                    fb = full * BATCH
                    lrow = wk_loc.at[full]
                    for g in range(8):
                        s_src = pl.ds(fb + g * 16, 16)
                        s_dst = pl.ds(g * 16, 16)
                        wk_col[s_dst] = wk_col[s_src]
                        wk_val[s_dst] = wk_val[s_src]
                        wk_loc[0, s_dst] = lrow[s_dst]

                return off & (BATCH - 1)

            stage(0, st0, ssem0).start()

            def pair_blocks(p, off):
                off = block1(2 * p, st0, ssem0, st1, ssem1, off)
                off = block1(2 * p + 1, st1, ssem1, st0, ssem0, off)
                return off

            off_end = lax.fori_loop(0, nblk // 2, pair_blocks, jnp.int32(0))

            a = (off_end >> 4) << 4
            for g in range(9):
                idxs = a + g * 16 + lane
                plsc.store_scatter(wk_val, [idxs], zeros16,
                                   mask=idxs >= off_end)
            run_batches(z_h, (off_end > 0).astype(jnp.int32))

        plsc.subcore_barrier()
        for t in range(SUB // 128):
            pltpu.sync_copy(acc.at[pl.ds(wb + t * 128, 128)],
                            out_ref.at[pl.ds(lo + wb + t * 128, 128)])
        plsc.subcore_barrier()


def _spmm_sc(n_out_pad, lists, zero_hbm):
    n_chunks = n_out_pad // CHUNK
    nblks = tuple(l[1] for l in lists)
    nl = len(lists)
    mesh = plsc.VectorSubcoreMesh(core_axis_name="c", subcore_axis_name="s",
                                  num_cores=NC, num_subcores=NS)

    def body(*refs):
        z_refs = refs[0:nl]
        pk_refs = refs[nl:2 * nl]
        zero_ref = refs[2 * nl]
        out_ref = refs[2 * nl + 1]
        scratch = refs[2 * nl + 2:]
        _spmm_body(n_chunks, nblks, z_refs, pk_refs, zero_ref, out_ref,
                   *scratch)

    kern = pl.kernel(
        body,
        out_type=jax.ShapeDtypeStruct((n_out_pad, C), jnp.float32),
        mesh=mesh,
        scratch_types=[
            pltpu.VMEM_SHARED((CHUNK, C), jnp.float32),
            pltpu.VMEM((32, C), jnp.float32),
            pltpu.VMEM((3 * SCAN,), jnp.int32),
            pltpu.VMEM((3 * SCAN,), jnp.int32),
            pltpu.VMEM((WKN,), jnp.int32),
            pltpu.VMEM((WKN // BATCH + 2, BATCH), jnp.int32),
            pltpu.VMEM((WKN,), jnp.float32),
            pltpu.VMEM((BATCH, C), jnp.float32),
            pltpu.VMEM((BATCH, C), jnp.float32),
            pltpu.SemaphoreType.DMA,
            pltpu.SemaphoreType.DMA,
            pltpu.SemaphoreType.DMA,
            pltpu.SemaphoreType.DMA,
        ],
        compiler_params=_compiler_params(),
    )
    args = [l[2] for l in lists] + [l[0] for l in lists] + [zero_hbm]
    return kern(*args)


def kernel(x0, x1, x2,
           inc1_row, inc1_col, inc1_val,
           inc2_row, inc2_col, inc2_val,
           h0_row, h0_col, h0_val,
           h1_row, h1_col, h1_val,
           h2_row, h2_col, h2_val,
           xbel0, xbel1, xbel2,
           W_same, W_l2h, W_h2l, RW, Rb):
    f32, i32 = jnp.float32, jnp.int32
    unit = NS * SCAN * 2

    def pad_list(row, col, val):
        n = row.shape[0]
        n_pad = -(-n // unit) * unit
        r = _pad_to(row.astype(i32), n_pad, PAD_ROW).reshape(-1, 1, SCAN)
        c = _pad_to(col.astype(i32), n_pad, 0).reshape(-1, 1, SCAN)
        v = lax.bitcast_convert_type(
            _pad_to(val.astype(f32), n_pad, 0.0), i32).reshape(-1, 1, SCAN)
        packed = jnp.concatenate([r, c, v], axis=1).reshape(-1)
        return (packed, n_pad // (NS * SCAN))

    h0 = pad_list(h0_row, h0_col, h0_val)
    h1 = pad_list(h1_row, h1_col, h1_val)
    h2 = pad_list(h2_row, h2_col, h2_val)
    i1 = pad_list(inc1_row, inc1_col, inc1_val)
    i1t = pad_list(inc1_col, inc1_row, inc1_val)
    i2 = pad_list(inc2_row, inc2_col, inc2_val)
    i2t = pad_list(inc2_col, inc2_row, inc2_val)

    zero_hbm = jnp.zeros((32, C), f32)

    a0, a1, a2 = x0, x1, x2
    for l in range(N_LAYERS):
        sig = l > 0
        z0s, z0l = _matmul_multi(a0, jnp.stack([W_same[l, 0], W_l2h[l, 0]]),
                                 N0, sig)
        z1s, z1h, z1l = _matmul_multi(
            a1, jnp.stack([W_same[l, 1], W_h2l[l, 0], W_l2h[l, 1]]), N1, sig)
        z2s, z2h = _matmul_multi(a2, jnp.stack([W_same[l, 2], W_h2l[l, 1]]),
                                 N2, sig)

        a0 = _spmm_sc(NP0, [h0 + (z0s,), i1 + (z1h,)], zero_hbm)
        a1 = _spmm_sc(NP1, [h1 + (z1s,), i2 + (z2h,), i1t + (z0l,)], zero_hbm)
        a2 = _spmm_sc(NP2, [h2 + (z2s,), i2t + (z1l,)], zero_hbm)

    p0 = _pool(a0, xbel0.astype(i32).reshape(N0, 1), N0)
    p1 = _pool(a1, xbel1.astype(i32).reshape(N1, 1), N1)
    p2 = _pool(a2, xbel2.astype(i32).reshape(N2, 1), N2)

    return _combine(jnp.stack([p0, p1, p2]), RW.astype(f32), Rb.astype(f32))


# speedup vs baseline: 6.8533x; 1.1112x over previous
import dataclasses
import functools

import jax
import jax.numpy as jnp
from jax import lax
from jax.experimental import pallas as pl
from jax.experimental.pallas import tpu as pltpu
from jax.experimental.pallas import tpu_sc as plsc

N0, N1, N2 = 50000, 150000, 100000
C = 128
NG = 8
N_LAYERS = 2

NC, NS = 2, 16
CHUNK = 10240
SUB = CHUNK // NS
SCAN = 1024
BATCH = 128
PAD_ROW = 1 << 30

NP0 = 5 * CHUNK
NP1 = 15 * CHUNK
NP2 = 10 * CHUNK

MM_BLK = 1000


def _compiler_params():
    cp = pltpu.CompilerParams()
    if "needs_layout_passes" in pltpu.CompilerParams.__dataclass_fields__:
        cp = dataclasses.replace(cp, needs_layout_passes=False)
    return cp


def _mm_body(m, sig, x_ref, w_ref, *out_refs):
    x = x_ref[...]
    if sig:
        x = 1.0 / (1.0 + jnp.exp(-x))
    for j in range(m):
        out_refs[j][...] = jnp.dot(x, w_ref[j],
                                   preferred_element_type=jnp.float32)


def _matmul_multi(x, ws, n, sig):
    m = ws.shape[0]
    return pl.pallas_call(
        functools.partial(_mm_body, m, sig),
        grid=(n // MM_BLK,),
        in_specs=[
            pl.BlockSpec((MM_BLK, C), lambda i: (i, 0)),
            pl.BlockSpec((m, C, C), lambda i: (0, 0, 0)),
        ],
        out_specs=[pl.BlockSpec((MM_BLK, C), lambda i: (i, 0))] * m,
        out_shape=[jax.ShapeDtypeStruct((n, C), jnp.float32)] * m,
    )(x, ws)


def _pool_body(x_ref, ids_ref, out_ref):
    @pl.when(pl.program_id(0) == 0)
    def _():
        out_ref[...] = jnp.zeros_like(out_ref)

    x = x_ref[...]
    x = 1.0 / (1.0 + jnp.exp(-x))
    ids = ids_ref[...]
    g = lax.broadcasted_iota(jnp.int32, (MM_BLK, NG), 1)
    onehot = (ids == g).astype(jnp.float32)
    out_ref[...] += lax.dot_general(
        onehot, x, (((0,), (0,)), ((), ())),
        preferred_element_type=jnp.float32)


def _pool(y, ids, n):
    return pl.pallas_call(
        _pool_body,
        grid=(n // MM_BLK,),
        in_specs=[
            pl.BlockSpec((MM_BLK, C), lambda i: (i, 0)),
            pl.BlockSpec((MM_BLK, 1), lambda i: (i, 0)),
        ],
        out_specs=pl.BlockSpec((NG, C), lambda i: (0, 0)),
        out_shape=jax.ShapeDtypeStruct((NG, C), jnp.float32),
    )(y, ids)


def _combine_body(p_ref, rw_ref, rb_ref, o_ref):
    acc = jnp.zeros((NG, 3), jnp.float32)
    for r in range(3):
        acc = acc + jnp.dot(p_ref[r], rw_ref[r],
                            preferred_element_type=jnp.float32)
    o_ref[...] = acc + jnp.sum(rb_ref[...], axis=0, keepdims=True)


def _combine(pooled, rw, rb):
    return pl.pallas_call(
        _combine_body,
        out_shape=jax.ShapeDtypeStruct((NG, 3), jnp.float32),
    )(pooled, rw, rb)


def _pad_to(a, n_pad, fill):
    return jnp.concatenate(
        [a, jnp.full((n_pad - a.shape[0],), fill, a.dtype)])


def _scale_rows(rows, wk_val, b):
    @pl.loop(0, BATCH, step=4)
    def _(r):
        base = b * BATCH + r
        sps = [plsc.load_gather(wk_val,
                                [jnp.full((16,), base + k, jnp.int32)])
               for k in range(4)]
        for k in range(4):
            rr = rows.at[r + k]
            for cc in range(8):
                s = pl.ds(cc * 16, 16)
                rr[s] = rr[s] * sps[k]


WKN = SCAN + BATCH + 32


def _spmm_body(n_chunks, nblks, z_refs, pk_refs, zero_hbm, out_ref,
               acc, zbuf, st0, st1, wk_col, wk_loc, wk_val, sums, gbase,
               rows0, rows1, ssem0, ssem1, gsem0, gsem1):
    core = lax.axis_index("c")
    sub = lax.axis_index("s")
    zeros16 = jnp.zeros((16,), jnp.float32)
    izeros16 = jnp.zeros((16,), jnp.int32)
    lane = lax.iota(jnp.int32, 16)

    pltpu.sync_copy(zero_hbm, zbuf)

    @pl.loop(0, WKN // 16)
    def _(j):
        wk_val[pl.ds(j * 16, 16)] = zeros16
        wk_col[pl.ds(j * 16, 16)] = izeros16

    @pl.loop(0, WKN // BATCH + 1)
    def _(j):
        for cc in range(8):
            wk_loc[j, pl.ds(cc * 16, 16)] = izeros16

    wb = sub * SUB

    def compact(st, lo, off0):
        ucap = jnp.uint32(CHUNK)

        @plsc.parallel_loop(0, SCAN // 16, 1, unroll=2)
        def _(j):
            rv = st[pl.ds(j * 16, 16)]
            mi = ((rv - lo).astype(jnp.uint32) < ucap).astype(jnp.int32)
            cs = plsc.cumsum(mi)
            plsc.store_scatter(sums, [jnp.full((16,), j, jnp.int32)], cs,
                               mask=lane == 15)

        off = off0
        for k in range(SCAN // 256):
            sk = sums[pl.ds(k * 16, 16)]
            ck = plsc.cumsum(sk)
            bk = off + ck - sk
            gbase[pl.ds(k * 16, 16)] = bk
            off = off + lax.squeeze(lax.slice(ck, [15], [16]), [0])

        @plsc.parallel_loop(0, SCAN // 16, 1, unroll=2)
        def _(j):
            rv = st[pl.ds(j * 16, 16)]
            d = rv - lo
            m = d.astype(jnp.uint32) < ucap
            mi = m.astype(jnp.int32)
            cv = st[pl.ds(SCAN + j * 16, 16)]
            vv = plsc.bitcast(st[pl.ds(2 * SCAN + j * 16, 16)], jnp.float32)
            base = plsc.load_gather(gbase, [jnp.full((16,), j, jnp.int32)])
            pos = base + plsc.cumsum(mi) - 1
            plsc.store_scatter(wk_col, [pos], cv, mask=m)
            plsc.store_scatter(wk_val, [pos], vv, mask=m)
            plsc.store_scatter(wk_loc, [pos >> 7, pos & 127], d, mask=m)

        return off

    def run_batches(z_h, nb):
        def idx(b):
            return wk_col.at[pl.ds(b * BATCH, BATCH)]

        @pl.when(nb > 0)
        def _():
            pltpu.async_copy(z_h.at[idx(0)], rows0, gsem0)

        def pair(pi, carry):
            b0 = 2 * pi
            b1 = b0 + 1
            pltpu.make_async_copy(z_h.at[idx(b0)], rows0, gsem0).wait()

            @pl.when(b1 < nb)
            def _():
                pltpu.async_copy(z_h.at[idx(b1)], rows1, gsem1)

            _scale_rows(rows0, wk_val, b0)
            pltpu.sync_copy(rows0, acc.at[wk_loc.at[b0]], add=True)

            @pl.when(b1 < nb)
            def _():
                pltpu.make_async_copy(z_h.at[idx(b1)], rows1, gsem1).wait()

                @pl.when(b1 + 1 < nb)
                def _():
                    pltpu.async_copy(z_h.at[idx(b1 + 1)], rows0, gsem0)

                _scale_rows(rows1, wk_val, b1)
                pltpu.sync_copy(rows1, acc.at[wk_loc.at[b1]], add=True)

            return carry

        lax.fori_loop(0, (nb + 1) // 2, pair, jnp.int32(0))

    @pl.loop(0, (n_chunks + 1 - core) // NC)
    def _(ci):
        chunk = ci * NC + core
        lo = chunk * CHUNK

        for t in range(SUB // 32):
            pltpu.sync_copy(zbuf, acc.at[pl.ds(wb + t * 32, 32)])
        plsc.subcore_barrier()

        for li in range(len(nblks)):
            z_h = z_refs[li]
            pk = pk_refs[li]
            nblk = nblks[li]
            ubase = sub * nblk

            def stage(b, st, sem):
                return pltpu.make_async_copy(
                    pk.at[pl.ds((ubase + b) * (3 * SCAN), 3 * SCAN)], st, sem)

            def block1(b, st, sem, stN, semN, off):
                stage(b, st, sem).wait()

                @pl.when(b + 1 < nblk)
                def _():
                    stage(b + 1, stN, semN).start()

                off = compact(st, lo, off)
                full = off >> 7
                run_batches(z_h, full)

                @pl.when(full > 0)
                def _():
                    fb = full * BATCH
                    lrow = wk_loc.at[full]
                    for g in range(8):
                        s_src = pl.ds(fb + g * 16, 16)
                        s_dst = pl.ds(g * 16, 16)
                        wk_col[s_dst] = wk_col[s_src]
                        wk_val[s_dst] = wk_val[s_src]
                        wk_loc[0, s_dst] = lrow[s_dst]

                return off & (BATCH - 1)

            stage(0, st0, ssem0).start()

            def pair_blocks(p, off):
                off = block1(2 * p, st0, ssem0, st1, ssem1, off)
                off = block1(2 * p + 1, st1, ssem1, st0, ssem0, off)
                return off

            off_end = lax.fori_loop(0, nblk // 2, pair_blocks, jnp.int32(0))

            a = (off_end >> 4) << 4
            for g in range(9):
                idxs = a + g * 16 + lane
                plsc.store_scatter(wk_val, [idxs], zeros16,
                                   mask=idxs >= off_end)
            run_batches(z_h, (off_end > 0).astype(jnp.int32))

        plsc.subcore_barrier()
        for t in range(SUB // 128):
            pltpu.sync_copy(acc.at[pl.ds(wb + t * 128, 128)],
                            out_ref.at[pl.ds(lo + wb + t * 128, 128)])
        plsc.subcore_barrier()


def _spmm_sc(n_out_pad, lists, zero_hbm):
    n_chunks = n_out_pad // CHUNK
    nblks = tuple(l[1] for l in lists)
    nl = len(lists)
    mesh = plsc.VectorSubcoreMesh(core_axis_name="c", subcore_axis_name="s",
                                  num_cores=NC, num_subcores=NS)

    def body(*refs):
        z_refs = refs[0:nl]
        pk_refs = refs[nl:2 * nl]
        zero_ref = refs[2 * nl]
        out_ref = refs[2 * nl + 1]
        scratch = refs[2 * nl + 2:]
        _spmm_body(n_chunks, nblks, z_refs, pk_refs, zero_ref, out_ref,
                   *scratch)

    kern = pl.kernel(
        body,
        out_type=jax.ShapeDtypeStruct((n_out_pad, C), jnp.float32),
        mesh=mesh,
        scratch_types=[
            pltpu.VMEM_SHARED((CHUNK, C), jnp.float32),
            pltpu.VMEM((32, C), jnp.float32),
            pltpu.VMEM((3 * SCAN,), jnp.int32),
            pltpu.VMEM((3 * SCAN,), jnp.int32),
            pltpu.VMEM((WKN,), jnp.int32),
            pltpu.VMEM((WKN // BATCH + 2, BATCH), jnp.int32),
            pltpu.VMEM((WKN,), jnp.float32),
            pltpu.VMEM((SCAN // 16,), jnp.int32),
            pltpu.VMEM((SCAN // 16,), jnp.int32),
            pltpu.VMEM((BATCH, C), jnp.float32),
            pltpu.VMEM((BATCH, C), jnp.float32),
            pltpu.SemaphoreType.DMA,
            pltpu.SemaphoreType.DMA,
            pltpu.SemaphoreType.DMA,
            pltpu.SemaphoreType.DMA,
        ],
        compiler_params=_compiler_params(),
    )
    args = [l[2] for l in lists] + [l[0] for l in lists] + [zero_hbm]
    return kern(*args)


def kernel(x0, x1, x2,
           inc1_row, inc1_col, inc1_val,
           inc2_row, inc2_col, inc2_val,
           h0_row, h0_col, h0_val,
           h1_row, h1_col, h1_val,
           h2_row, h2_col, h2_val,
           xbel0, xbel1, xbel2,
           W_same, W_l2h, W_h2l, RW, Rb):
    f32, i32 = jnp.float32, jnp.int32
    unit = NS * SCAN * 2

    def pad_list(row, col, val):
        n = row.shape[0]
        n_pad = -(-n // unit) * unit
        r = _pad_to(row.astype(i32), n_pad, PAD_ROW).reshape(-1, 1, SCAN)
        c = _pad_to(col.astype(i32), n_pad, 0).reshape(-1, 1, SCAN)
        v = lax.bitcast_convert_type(
            _pad_to(val.astype(f32), n_pad, 0.0), i32).reshape(-1, 1, SCAN)
        packed = jnp.concatenate([r, c, v], axis=1).reshape(-1)
        return (packed, n_pad // (NS * SCAN))

    h0 = pad_list(h0_row, h0_col, h0_val)
    h1 = pad_list(h1_row, h1_col, h1_val)
    h2 = pad_list(h2_row, h2_col, h2_val)
    i1 = pad_list(inc1_row, inc1_col, inc1_val)
    i1t = pad_list(inc1_col, inc1_row, inc1_val)
    i2 = pad_list(inc2_row, inc2_col, inc2_val)
    i2t = pad_list(inc2_col, inc2_row, inc2_val)

    zero_hbm = jnp.zeros((32, C), f32)

    a0, a1, a2 = x0, x1, x2
    for l in range(N_LAYERS):
        sig = l > 0
        z0s, z0l = _matmul_multi(a0, jnp.stack([W_same[l, 0], W_l2h[l, 0]]),
                                 N0, sig)
        z1s, z1h, z1l = _matmul_multi(
            a1, jnp.stack([W_same[l, 1], W_h2l[l, 0], W_l2h[l, 1]]), N1, sig)
        z2s, z2h = _matmul_multi(a2, jnp.stack([W_same[l, 2], W_h2l[l, 1]]),
                                 N2, sig)

        a0 = _spmm_sc(NP0, [h0 + (z0s,), i1 + (z1h,)], zero_hbm)
        a1 = _spmm_sc(NP1, [h1 + (z1s,), i2 + (z2h,), i1t + (z0l,)], zero_hbm)
        a2 = _spmm_sc(NP2, [h2 + (z2s,), i2t + (z1l,)], zero_hbm)

    p0 = _pool(a0, xbel0.astype(i32).reshape(N0, 1), N0)
    p1 = _pool(a1, xbel1.astype(i32).reshape(N1, 1), N1)
    p2 = _pool(a2, xbel2.astype(i32).reshape(N2, 1), N2)

    return _combine(jnp.stack([p0, p1, p2]), RW.astype(f32), Rb.astype(f32))
